# baseline (speedup 1.0000x reference)
import functools

import numpy as np
import jax
from jax import lax
import jax.numpy as jnp
from jax.experimental import pallas as pl
from jax.experimental.pallas import tpu as pltpu
from jax.experimental.pallas import tpu_sc as plsc

H = 1024; I = 512; NH = 16; DQ = 384; DKV = 128; DH = 64; DR = 32
E = 16; K = 2; MAXLEN = 4096; S = 2048; B = 1
EPS = 1.1920929e-07
MB = 256
GT = 128
NP = S * K
PAD = NP + E * GT
TMAX = NP // GT + E


def _rope_tables():
    inv_freq = 1.0 / (10000.0 ** (np.arange(0, DR, 2, dtype=np.float32) / DR))
    t = np.arange(S, dtype=np.float32)
    freqs = np.outer(t, inv_freq)
    emb = np.concatenate([freqs, freqs], axis=-1)
    return jnp.asarray(np.cos(emb)), jnp.asarray(np.sin(emb))


def _rms(x, w):
    return x * jax.lax.rsqrt(jnp.mean(x * x, axis=-1, keepdims=True) + EPS) * w


def _silu(x):
    return x * jax.nn.sigmoid(x)


def _dot(a, b):
    return jnp.dot(a, b, preferred_element_type=jnp.float32)


def _mla_kernel(hs_ref, inw_ref, wdq_ref, nqw_ref, wuqx_ref, wdkvx_ref,
                nkvw_ref, wukvx_ref, cos_ref, sin_ref, cos16_ref, sin16_ref,
                o_ref):
    bf = jnp.bfloat16
    f32 = jnp.float32
    x = _rms(hs_ref[...], inw_ref[...]).astype(bf)
    ckv = _dot(x, wdkvx_ref[...])
    kv = _dot(_rms(ckv[:, :DKV], nkvw_ref[...]).astype(bf), wukvx_ref[...])
    k_c_all = kv[:, :NH * DH].astype(bf)
    v_all = kv[:, NH * DH:].astype(bf)
    k_r = (ckv[:, DKV:DKV + DR] * cos_ref[...]
           + ckv[:, DKV + DR:] * sin_ref[...]).astype(bf)
    cq = _dot(x, wdq_ref[...])
    qx = _dot(_rms(cq, nqw_ref[...]).astype(bf), wuqx_ref[...])
    q_c_all = qx[:, :NH * DH].astype(bf)
    q_r_all = (qx[:, NH * DH:NH * (DH + DR)] * cos16_ref[...].astype(f32)
               + qx[:, NH * (DH + DR):] * sin16_ref[...].astype(f32)).astype(bf)
    scale = 1.0 / np.sqrt(np.float32(DH + DR))
    RB = S // 4
    for h in range(NH):
        k_c = k_c_all[:, h * DH:(h + 1) * DH]
        v = v_all[:, h * DH:(h + 1) * DH]
        for r in range(4):
            rows = slice(r * RB, (r + 1) * RB)
            q_c = q_c_all[rows, h * DH:(h + 1) * DH]
            q_r = q_r_all[rows, h * DR:(h + 1) * DR]
            s = (_dot(q_c, k_c.T) + _dot(q_r, k_r.T)) * scale
            m = jnp.max(s, axis=-1, keepdims=True)
            p = jnp.exp(s - m)
            p = (p / jnp.sum(p, axis=-1, keepdims=True)).astype(bf)
            o_ref[rows, h * DH:(h + 1) * DH] = _dot(p, v).astype(bf)


def _post_kernel(o_ref, hs_ref, wo_ref, pnw_ref, wgs_ref, wus_ref, wds_ref,
                 wg_ref, ybase_ref, x2_ref, gate_ref, sel_ref):
    attn_out = _dot(o_ref[...], wo_ref[...]) + hs_ref[...]
    x2 = _rms(attn_out, pnw_ref[...])
    x2_ref[...] = x2
    xb = x2.astype(jnp.bfloat16)
    shared = _dot((_silu(_dot(xb, wgs_ref[...]))
                   * _dot(xb, wus_ref[...])).astype(jnp.bfloat16),
                  wds_ref[...])
    ybase_ref[...] = attn_out + shared
    scores = jax.nn.sigmoid(_dot(x2, wg_ref[...]))
    lane = jax.lax.broadcasted_iota(jnp.int32, scores.shape, 1)
    m1 = jnp.max(scores, axis=-1, keepdims=True)
    i1 = jnp.min(jnp.where(scores >= m1, lane, E), axis=-1, keepdims=True)
    first1 = lane == i1
    masked = jnp.where(first1, -jnp.inf, scores)
    m2 = jnp.max(masked, axis=-1, keepdims=True)
    i2 = jnp.min(jnp.where(masked >= m2, lane, E), axis=-1, keepdims=True)
    first2 = lane == i2
    denom = m1 + m2
    gate_ref[...] = jnp.where(first1, m1 / denom, 0.0) + \
        jnp.where(first2, m2 / denom, 0.0)
    sel_ref[...] = (first1 | first2).astype(jnp.float32)


def _routing_kernel(gate_ref, sel_ref, dlo_ref, dhi_ref, glo_ref, ghi_ref,
                    te_ref, nt_ref):
    g = gate_ref[...]
    sel = sel_ref[...]
    rank = sel * 0.0
    counts = jnp.sum(sel, axis=0, keepdims=True)
    tiles_e = jnp.floor((counts + (GT - 1)) * (1.0 / GT))
    ui = lax.broadcasted_iota(jnp.int32, (E, E), 0)
    uj = lax.broadcasted_iota(jnp.int32, (E, E), 1)
    ustrict = (ui < uj).astype(jnp.float32)
    tile_off = _dot(tiles_e, ustrict)
    off_rows = tile_off * float(GT)
    dmat = off_rows + rank
    lane = lax.broadcasted_iota(jnp.int32, (S, E), 1)
    lanef = lane.astype(jnp.float32)
    e_lo = jnp.min(jnp.where(sel > 0, lanef, float(E)), axis=-1,
                   keepdims=True)
    e_hi = jnp.max(jnp.where(sel > 0, lanef, -1.0), axis=-1, keepdims=True)
    sel_lo = (lanef == e_lo).astype(jnp.float32)
    sel_hi = (lanef == e_hi).astype(jnp.float32)
    dlo_ref[...] = jnp.sum(dmat * sel_lo, axis=-1,
                           keepdims=True).astype(jnp.int32)
    dhi_ref[...] = jnp.sum(dmat * sel_hi, axis=-1,
                           keepdims=True).astype(jnp.int32)
    glo_ref[...] = jnp.sum(g * sel_lo, axis=-1, keepdims=True)
    ghi_ref[...] = jnp.sum(g * sel_hi, axis=-1, keepdims=True)
    jcol = lax.broadcasted_iota(jnp.int32, (64, E), 0).astype(jnp.float32)
    offb = jnp.broadcast_to(tile_off, (64, E))
    te_ref[...] = (jnp.sum((offb <= jcol).astype(jnp.float32), axis=-1,
                           keepdims=True) - 1.0).astype(jnp.int32)
    nt_ref[...] = jnp.sum(tiles_e, axis=-1, keepdims=True).astype(jnp.int32)


def _grouped_ffn_kernel(te_ref, nt_ref, x_ref, wge_ref, wue_ref, wde_ref,
                        y_ref):
    j = pl.program_id(0)

    @pl.when(j < nt_ref[0])
    def _():
        x = x_ref[...].astype(jnp.bfloat16)
        h = _silu(_dot(x, wge_ref[0])) * _dot(x, wue_ref[0])
        y_ref[...] = _dot(h.astype(jnp.bfloat16), wde_ref[0])


_SC_INFO = None


def _sc_info():
    global _SC_INFO
    if _SC_INFO is None:
        info = plsc.get_sparse_core_info()
        _SC_INFO = (info.num_cores, info.num_subcores)
    return _SC_INFO


def _sc_scatter_rows(x2, d_all):
    nc, ns = _sc_info()
    nw = nc * ns
    rows_w = NP // nw
    chunk = rows_w // 2
    mesh = plsc.VectorSubcoreMesh(core_axis_name="c", subcore_axis_name="s")

    @functools.partial(
        pl.kernel, mesh=mesh,
        out_type=jax.ShapeDtypeStruct((PAD, H), jnp.float32),
        scratch_types=[
            pltpu.VMEM((chunk,), jnp.int32),
            pltpu.VMEM((chunk, H), jnp.float32),
            pltpu.SemaphoreType.DMA,
        ],
    )
    def scatter_k(x2_hbm, idx_hbm, out_hbm, idx_v, rows_v, sem):
        wid = lax.axis_index("s") * nc + lax.axis_index("c")
        for c in range(2):
            ib = wid * rows_w + c * chunk
            sb = (wid % ns) * rows_w + c * chunk
            pltpu.sync_copy(idx_hbm.at[pl.ds(ib, chunk)], idx_v)
            pltpu.sync_copy(x2_hbm.at[pl.ds(sb, chunk)], rows_v)
            pltpu.async_copy(rows_v, out_hbm.at[idx_v], sem).wait()

    return scatter_k(x2, d_all)


def _sc_gather_rows(ys, d_all):
    nc, ns = _sc_info()
    nw = nc * ns
    rows_w = NP // nw
    chunk = rows_w // 2
    mesh = plsc.VectorSubcoreMesh(core_axis_name="c", subcore_axis_name="s")

    @functools.partial(
        pl.kernel, mesh=mesh,
        out_type=jax.ShapeDtypeStruct((NP, H), jnp.float32),
        scratch_types=[
            pltpu.VMEM((chunk,), jnp.int32),
            pltpu.VMEM((chunk, H), jnp.float32),
            pltpu.SemaphoreType.DMA,
        ],
    )
    def gather_k(ys_hbm, idx_hbm, out_hbm, idx_v, rows_v, sem):
        wid = lax.axis_index("s") * nc + lax.axis_index("c")
        for c in range(2):
            ib = wid * rows_w + c * chunk
            pltpu.sync_copy(idx_hbm.at[pl.ds(ib, chunk)], idx_v)
            pltpu.async_copy(ys_hbm.at[idx_v], rows_v, sem).wait()
            pltpu.sync_copy(rows_v, out_hbm.at[pl.ds(ib, chunk)])

    return gather_k(ys, d_all)


def _combine_kernel(ybase_ref, y1_ref, y2_ref, glo_ref, ghi_ref, out_ref):
    out_ref[...] = (ybase_ref[...] + glo_ref[...] * y1_ref[...]
                    + ghi_ref[...] * y2_ref[...])


def kernel(hidden_states, input_norm_w, post_norm_w, W_dq, norm_q_w, W_uq,
           W_dkv, norm_kv_w, W_ukv, W_o, W_gate, Wg_shared, Wu_shared,
           Wd_shared, Wg_experts, Wu_experts, Wd_experts):
    hs = hidden_states.reshape(S, H)
    cos, sin = _rope_tables()
    f32 = jnp.float32

    bf = jnp.bfloat16
    wdq_b = W_dq.astype(bf)
    wuq3 = W_uq.reshape(DQ, NH, DH + DR)
    wuq_rot = jnp.concatenate([-wuq3[:, :, DH + DR // 2:],
                               wuq3[:, :, DH:DH + DR // 2]], axis=2)
    wuqx_b = jnp.concatenate([
        wuq3[:, :, :DH].reshape(DQ, NH * DH),
        wuq3[:, :, DH:].reshape(DQ, NH * DR),
        wuq_rot.reshape(DQ, NH * DR)], axis=1).astype(bf)
    wdkvx_b = jnp.concatenate([
        W_dkv,
        -W_dkv[:, DKV + DR // 2:],
        W_dkv[:, DKV:DKV + DR // 2]], axis=1).astype(bf)
    wukv3 = W_ukv.reshape(DKV, NH, 2 * DH)
    wukvx_b = jnp.concatenate([
        wukv3[:, :, :DH].reshape(DKV, NH * DH),
        wukv3[:, :, DH:].reshape(DKV, NH * DH)], axis=1).astype(bf)
    cos16 = jnp.tile(cos, (1, NH)).astype(bf)
    sin16 = jnp.tile(sin, (1, NH)).astype(bf)
    wo_b = W_o.astype(bf)
    wgs_b = Wg_shared.astype(bf)
    wus_b = Wu_shared.astype(bf)
    wds_b = Wd_shared.astype(bf)
    wge_b = Wg_experts.astype(bf)
    wue_b = Wu_experts.astype(bf)
    wde_b = Wd_experts.astype(bf)
    inw = input_norm_w.reshape(1, H)
    nqw = norm_q_w.reshape(1, DQ)
    nkvw = norm_kv_w.reshape(1, DKV)
    pnw = post_norm_w.reshape(1, H)
    nm = S // MB

    full = lambda r, c: pl.BlockSpec((r, c), lambda: (0, 0))
    o_flat = pl.pallas_call(
        _mla_kernel,
        in_specs=[
            full(S, H), full(1, H), full(H, DQ), full(1, DQ),
            full(DQ, NH * (DH + 2 * DR)), full(H, DKV + 2 * DR),
            full(1, DKV), full(DKV, NH * 2 * DH),
            full(S, DR), full(S, DR), full(S, NH * DR), full(S, NH * DR),
        ],
        out_specs=full(S, NH * DH),
        out_shape=jax.ShapeDtypeStruct((S, NH * DH), bf),
    )(hs, inw, wdq_b, nqw, wuqx_b, wdkvx_b, nkvw, wukvx_b,
      cos, sin, cos16, sin16)

    y_base, x2, gate_dense, sel_mask = pl.pallas_call(
        _post_kernel,
        grid=(nm,),
        in_specs=[
            pl.BlockSpec((MB, NH * DH), lambda m: (m, 0)),
            pl.BlockSpec((MB, H), lambda m: (m, 0)),
            pl.BlockSpec((NH * DH, H), lambda m: (0, 0)),
            pl.BlockSpec((1, H), lambda m: (0, 0)),
            pl.BlockSpec((H, I), lambda m: (0, 0)),
            pl.BlockSpec((H, I), lambda m: (0, 0)),
            pl.BlockSpec((I, H), lambda m: (0, 0)),
            pl.BlockSpec((H, E), lambda m: (0, 0)),
        ],
        out_specs=[
            pl.BlockSpec((MB, H), lambda m: (m, 0)),
            pl.BlockSpec((MB, H), lambda m: (m, 0)),
            pl.BlockSpec((MB, E), lambda m: (m, 0)),
            pl.BlockSpec((MB, E), lambda m: (m, 0)),
        ],
        out_shape=[
            jax.ShapeDtypeStruct((S, H), f32),
            jax.ShapeDtypeStruct((S, H), f32),
            jax.ShapeDtypeStruct((S, E), f32),
            jax.ShapeDtypeStruct((S, E), f32),
        ],
    )(o_flat, hs, wo_b, pnw, wgs_b, wus_b, wds_b, W_gate)

    i32 = jnp.int32
    d_lo, d_hi, g_lo, g_hi, te64, ntile = pl.pallas_call(
        _routing_kernel,
        grid=(1,),
        in_specs=[
            pl.BlockSpec((S, E), lambda m: (0, 0)),
            pl.BlockSpec((S, E), lambda m: (0, 0)),
        ],
        out_specs=[
            pl.BlockSpec((S, 1), lambda m: (0, 0)),
            pl.BlockSpec((S, 1), lambda m: (0, 0)),
            pl.BlockSpec((S, 1), lambda m: (0, 0)),
            pl.BlockSpec((S, 1), lambda m: (0, 0)),
            pl.BlockSpec((64, 1), lambda m: (0, 0)),
            pl.BlockSpec((1, 1), lambda m: (0, 0)),
        ],
        out_shape=[
            jax.ShapeDtypeStruct((S, 1), i32),
            jax.ShapeDtypeStruct((S, 1), i32),
            jax.ShapeDtypeStruct((S, 1), f32),
            jax.ShapeDtypeStruct((S, 1), f32),
            jax.ShapeDtypeStruct((64, 1), i32),
            jax.ShapeDtypeStruct((1, 1), i32),
        ],
    )(gate_dense, sel_mask)

    d_all = jnp.concatenate([d_lo, d_hi], axis=0).reshape(NP)
    te = te64.reshape(64)
    nt = ntile.reshape(1)

    x_sorted = _sc_scatter_rows(x2, d_all)

    y_sorted = pl.pallas_call(
        _grouped_ffn_kernel,
        grid_spec=pltpu.PrefetchScalarGridSpec(
            num_scalar_prefetch=2,
            grid=(TMAX,),
            in_specs=[
                pl.BlockSpec((GT, H), lambda j, te, nt: (j, 0)),
                pl.BlockSpec((1, H, I), lambda j, te, nt: (te[j], 0, 0)),
                pl.BlockSpec((1, H, I), lambda j, te, nt: (te[j], 0, 0)),
                pl.BlockSpec((1, I, H), lambda j, te, nt: (te[j], 0, 0)),
            ],
            out_specs=pl.BlockSpec((GT, H), lambda j, te, nt: (j, 0)),
        ),
        out_shape=jax.ShapeDtypeStruct((PAD, H), f32),
    )(te, nt, x_sorted, wge_b, wue_b, wde_b)

    y_pairs = _sc_gather_rows(y_sorted, d_all)
    y1 = y_pairs[:S]
    y2 = y_pairs[S:]

    out = pl.pallas_call(
        _combine_kernel,
        grid=(nm,),
        in_specs=[
            pl.BlockSpec((MB, H), lambda m: (m, 0)),
            pl.BlockSpec((MB, H), lambda m: (m, 0)),
            pl.BlockSpec((MB, H), lambda m: (m, 0)),
            pl.BlockSpec((MB, 1), lambda m: (m, 0)),
            pl.BlockSpec((MB, 1), lambda m: (m, 0)),
        ],
        out_specs=pl.BlockSpec((MB, H), lambda m: (m, 0)),
        out_shape=jax.ShapeDtypeStruct((S, H), f32),
    )(y_base, y1, y2, g_lo, g_hi)

    return out.reshape(B, S, H)


# speedup vs baseline: 1.1214x; 1.1214x over previous
import functools

import numpy as np
import jax
from jax import lax
import jax.numpy as jnp
from jax.experimental import pallas as pl
from jax.experimental.pallas import tpu as pltpu
from jax.experimental.pallas import tpu_sc as plsc

H = 1024; I = 512; NH = 16; DQ = 384; DKV = 128; DH = 64; DR = 32
E = 16; K = 2; MAXLEN = 4096; S = 2048; B = 1
EPS = 1.1920929e-07
MB = 256
GT = 128
NP = S * K
PAD = NP + E * GT
TMAX = NP // GT + E


def _rope_tables():
    inv_freq = 1.0 / (10000.0 ** (np.arange(0, DR, 2, dtype=np.float32) / DR))
    t = np.arange(S, dtype=np.float32)
    freqs = np.outer(t, inv_freq)
    emb = np.concatenate([freqs, freqs], axis=-1)
    return jnp.asarray(np.cos(emb)), jnp.asarray(np.sin(emb))


def _rms(x, w):
    return x * jax.lax.rsqrt(jnp.mean(x * x, axis=-1, keepdims=True) + EPS) * w


def _silu(x):
    return x * jax.nn.sigmoid(x)


def _dot(a, b):
    return jnp.dot(a, b, preferred_element_type=jnp.float32)


def _mla_kernel(hs_ref, inw_ref, wdq_ref, nqw_ref, wuqx_ref, wdkvx_ref,
                nkvw_ref, wukvx_ref, cos_ref, sin_ref, cos16_ref, sin16_ref,
                o_ref):
    bf = jnp.bfloat16
    f32 = jnp.float32
    x = _rms(hs_ref[...], inw_ref[...]).astype(bf)
    ckv = _dot(x, wdkvx_ref[...])
    kv = _dot(_rms(ckv[:, :DKV], nkvw_ref[...]).astype(bf), wukvx_ref[...])
    k_c_all = kv[:, :NH * DH].astype(bf)
    v_all = kv[:, NH * DH:].astype(bf)
    k_r = (ckv[:, DKV:DKV + DR] * cos_ref[...]
           + ckv[:, DKV + DR:] * sin_ref[...]).astype(bf)
    cq = _dot(x, wdq_ref[...])
    qx = _dot(_rms(cq, nqw_ref[...]).astype(bf), wuqx_ref[...])
    q_c_all = qx[:, :NH * DH].astype(bf)
    q_r_all = (qx[:, NH * DH:NH * (DH + DR)] * cos16_ref[...].astype(f32)
               + qx[:, NH * (DH + DR):] * sin16_ref[...].astype(f32)).astype(bf)
    scale = 1.0 / np.sqrt(np.float32(DH + DR))
    RB = S // 4
    for h in range(NH):
        k_c = k_c_all[:, h * DH:(h + 1) * DH]
        v = v_all[:, h * DH:(h + 1) * DH]
        for r in range(4):
            rows = slice(r * RB, (r + 1) * RB)
            q_c = q_c_all[rows, h * DH:(h + 1) * DH]
            q_r = q_r_all[rows, h * DR:(h + 1) * DR]
            s = (_dot(q_c, k_c.T) + _dot(q_r, k_r.T)) * scale
            m = jnp.max(s, axis=-1, keepdims=True)
            p = jnp.exp(s - m)
            p = (p / jnp.sum(p, axis=-1, keepdims=True)).astype(bf)
            o_ref[rows, h * DH:(h + 1) * DH] = _dot(p, v).astype(bf)


def _post_kernel(o_ref, hs_ref, wo_ref, pnw_ref, wgs_ref, wus_ref, wds_ref,
                 wg_ref, ybase_ref, x2_ref, dlo_ref, dhi_ref, glo_ref,
                 ghi_ref, te_ref, nt_ref, gate_s, sel_s, rank_s, carry_s):
    m = pl.program_id(0)
    nm = S // MB

    @pl.when(m == 0)
    def _():
        carry_s[...] = jnp.zeros((1, E), jnp.float32)

    @pl.when(m < nm)
    def _():
        attn_out = _dot(o_ref[...], wo_ref[...]) + hs_ref[...]
        x2 = _rms(attn_out, pnw_ref[...])
        x2_ref[...] = x2
        xb = x2.astype(jnp.bfloat16)
        shared = _dot((_silu(_dot(xb, wgs_ref[...]))
                       * _dot(xb, wus_ref[...])).astype(jnp.bfloat16),
                      wds_ref[...])
        ybase_ref[...] = attn_out + shared
        scores = jax.nn.sigmoid(_dot(x2, wg_ref[...]))
        lane = jax.lax.broadcasted_iota(jnp.int32, scores.shape, 1)
        m1 = jnp.max(scores, axis=-1, keepdims=True)
        i1 = jnp.min(jnp.where(scores >= m1, lane, E), axis=-1, keepdims=True)
        first1 = lane == i1
        masked = jnp.where(first1, -jnp.inf, scores)
        m2 = jnp.max(masked, axis=-1, keepdims=True)
        i2 = jnp.min(jnp.where(masked >= m2, lane, E), axis=-1, keepdims=True)
        first2 = lane == i2
        denom = m1 + m2
        gate = jnp.where(first1, m1 / denom, 0.0) + \
            jnp.where(first2, m2 / denom, 0.0)
        sel = (first1 | first2).astype(jnp.float32)
        gate_s[pl.ds(m * MB, MB), :] = gate
        sel_s[pl.ds(m * MB, MB), :] = sel
        ri = lax.broadcasted_iota(jnp.int32, (MB, MB), 0)
        ci = lax.broadcasted_iota(jnp.int32, (MB, MB), 1)
        lstrict = (ri > ci).astype(jnp.bfloat16)
        rank_s[pl.ds(m * MB, MB), :] = (
            _dot(lstrict, sel.astype(jnp.bfloat16)) + carry_s[...])
        carry_s[...] += jnp.sum(sel, axis=0, keepdims=True)

    @pl.when(m == nm)
    def _():
        g = gate_s[...]
        sel = sel_s[...]
        rank = rank_s[...]
        counts = carry_s[...]
        tiles_e = jnp.floor((counts + (GT - 1)) * (1.0 / GT))
        ui = lax.broadcasted_iota(jnp.int32, (E, E), 0)
        uj = lax.broadcasted_iota(jnp.int32, (E, E), 1)
        ustrict = (ui < uj).astype(jnp.float32)
        tile_off = _dot(tiles_e, ustrict)
        dmat = tile_off * float(GT) + rank
        lane = lax.broadcasted_iota(jnp.int32, (S, E), 1)
        lanef = lane.astype(jnp.float32)
        e_lo = jnp.min(jnp.where(sel > 0, lanef, float(E)), axis=-1,
                       keepdims=True)
        e_hi = jnp.max(jnp.where(sel > 0, lanef, -1.0), axis=-1,
                       keepdims=True)
        sel_lo = (lanef == e_lo).astype(jnp.float32)
        sel_hi = (lanef == e_hi).astype(jnp.float32)
        dlo_ref[...] = jnp.sum(dmat * sel_lo, axis=-1,
                               keepdims=True).astype(jnp.int32)
        dhi_ref[...] = jnp.sum(dmat * sel_hi, axis=-1,
                               keepdims=True).astype(jnp.int32)
        glo_ref[...] = jnp.sum(g * sel_lo, axis=-1, keepdims=True)
        ghi_ref[...] = jnp.sum(g * sel_hi, axis=-1, keepdims=True)
        jcol = lax.broadcasted_iota(jnp.int32, (64, E), 0).astype(jnp.float32)
        offb = jnp.broadcast_to(tile_off, (64, E))
        te_ref[...] = (jnp.sum((offb <= jcol).astype(jnp.float32), axis=-1,
                               keepdims=True) - 1.0).astype(jnp.int32)
        nt_ref[...] = jnp.sum(tiles_e, axis=-1,
                              keepdims=True).astype(jnp.int32)


def _grouped_ffn_kernel(te_ref, nt_ref, x_ref, wge_ref, wue_ref, wde_ref,
                        y_ref):
    j = pl.program_id(0)

    @pl.when(j < nt_ref[0])
    def _():
        x = x_ref[...].astype(jnp.bfloat16)
        h = _silu(_dot(x, wge_ref[0])) * _dot(x, wue_ref[0])
        y_ref[...] = _dot(h.astype(jnp.bfloat16), wde_ref[0])


_SC_INFO = None


def _sc_info():
    global _SC_INFO
    if _SC_INFO is None:
        info = plsc.get_sparse_core_info()
        _SC_INFO = (info.num_cores, info.num_subcores)
    return _SC_INFO


def _sc_scatter_rows(x2, d_all):
    nc, ns = _sc_info()
    nw = nc * ns
    rows_w = NP // nw
    chunk = rows_w // 2
    mesh = plsc.VectorSubcoreMesh(core_axis_name="c", subcore_axis_name="s")

    @functools.partial(
        pl.kernel, mesh=mesh,
        out_type=jax.ShapeDtypeStruct((PAD, H), jnp.float32),
        scratch_types=[
            pltpu.VMEM((chunk,), jnp.int32),
            pltpu.VMEM((chunk, H), jnp.float32),
            pltpu.SemaphoreType.DMA,
        ],
    )
    def scatter_k(x2_hbm, idx_hbm, out_hbm, idx_v, rows_v, sem):
        wid = lax.axis_index("s") * nc + lax.axis_index("c")
        for c in range(2):
            ib = wid * rows_w + c * chunk
            sb = (wid % ns) * rows_w + c * chunk
            pltpu.sync_copy(idx_hbm.at[pl.ds(ib, chunk)], idx_v)
            pltpu.sync_copy(x2_hbm.at[pl.ds(sb, chunk)], rows_v)
            pltpu.async_copy(rows_v, out_hbm.at[idx_v], sem).wait()

    return scatter_k(x2, d_all)


def _sc_gather_rows(ys, d_all):
    nc, ns = _sc_info()
    nw = nc * ns
    rows_w = NP // nw
    chunk = rows_w // 2
    mesh = plsc.VectorSubcoreMesh(core_axis_name="c", subcore_axis_name="s")

    @functools.partial(
        pl.kernel, mesh=mesh,
        out_type=jax.ShapeDtypeStruct((NP, H), jnp.float32),
        scratch_types=[
            pltpu.VMEM((chunk,), jnp.int32),
            pltpu.VMEM((chunk, H), jnp.float32),
            pltpu.SemaphoreType.DMA,
        ],
    )
    def gather_k(ys_hbm, idx_hbm, out_hbm, idx_v, rows_v, sem):
        wid = lax.axis_index("s") * nc + lax.axis_index("c")
        for c in range(2):
            ib = wid * rows_w + c * chunk
            pltpu.sync_copy(idx_hbm.at[pl.ds(ib, chunk)], idx_v)
            pltpu.async_copy(ys_hbm.at[idx_v], rows_v, sem).wait()
            pltpu.sync_copy(rows_v, out_hbm.at[pl.ds(ib, chunk)])

    return gather_k(ys, d_all)


def _combine_kernel(ybase_ref, y1_ref, y2_ref, glo_ref, ghi_ref, out_ref):
    out_ref[...] = (ybase_ref[...] + glo_ref[...] * y1_ref[...]
                    + ghi_ref[...] * y2_ref[...])


def kernel(hidden_states, input_norm_w, post_norm_w, W_dq, norm_q_w, W_uq,
           W_dkv, norm_kv_w, W_ukv, W_o, W_gate, Wg_shared, Wu_shared,
           Wd_shared, Wg_experts, Wu_experts, Wd_experts):
    hs = hidden_states.reshape(S, H)
    cos, sin = _rope_tables()
    f32 = jnp.float32

    bf = jnp.bfloat16
    wdq_b = W_dq.astype(bf)
    wuq3 = W_uq.reshape(DQ, NH, DH + DR)
    wuq_rot = jnp.concatenate([-wuq3[:, :, DH + DR // 2:],
                               wuq3[:, :, DH:DH + DR // 2]], axis=2)
    wuqx_b = jnp.concatenate([
        wuq3[:, :, :DH].reshape(DQ, NH * DH),
        wuq3[:, :, DH:].reshape(DQ, NH * DR),
        wuq_rot.reshape(DQ, NH * DR)], axis=1).astype(bf)
    wdkvx_b = jnp.concatenate([
        W_dkv,
        -W_dkv[:, DKV + DR // 2:],
        W_dkv[:, DKV:DKV + DR // 2]], axis=1).astype(bf)
    wukv3 = W_ukv.reshape(DKV, NH, 2 * DH)
    wukvx_b = jnp.concatenate([
        wukv3[:, :, :DH].reshape(DKV, NH * DH),
        wukv3[:, :, DH:].reshape(DKV, NH * DH)], axis=1).astype(bf)
    cos16 = jnp.tile(cos, (1, NH)).astype(bf)
    sin16 = jnp.tile(sin, (1, NH)).astype(bf)
    wo_b = W_o.astype(bf)
    wgs_b = Wg_shared.astype(bf)
    wus_b = Wu_shared.astype(bf)
    wds_b = Wd_shared.astype(bf)
    wge_b = Wg_experts.astype(bf)
    wue_b = Wu_experts.astype(bf)
    wde_b = Wd_experts.astype(bf)
    inw = input_norm_w.reshape(1, H)
    nqw = norm_q_w.reshape(1, DQ)
    nkvw = norm_kv_w.reshape(1, DKV)
    pnw = post_norm_w.reshape(1, H)
    nm = S // MB

    full = lambda r, c: pl.BlockSpec((r, c), lambda: (0, 0))
    o_flat = pl.pallas_call(
        _mla_kernel,
        in_specs=[
            full(S, H), full(1, H), full(H, DQ), full(1, DQ),
            full(DQ, NH * (DH + 2 * DR)), full(H, DKV + 2 * DR),
            full(1, DKV), full(DKV, NH * 2 * DH),
            full(S, DR), full(S, DR), full(S, NH * DR), full(S, NH * DR),
        ],
        out_specs=full(S, NH * DH),
        out_shape=jax.ShapeDtypeStruct((S, NH * DH), bf),
    )(hs, inw, wdq_b, nqw, wuqx_b, wdkvx_b, nkvw, wukvx_b,
      cos, sin, cos16, sin16)

    i32 = jnp.int32
    nmc = nm - 1
    y_base, x2, d_lo, d_hi, g_lo, g_hi, te64, ntile = pl.pallas_call(
        _post_kernel,
        grid=(nm + 1,),
        in_specs=[
            pl.BlockSpec((MB, NH * DH), lambda m: (jnp.minimum(m, nmc), 0)),
            pl.BlockSpec((MB, H), lambda m: (jnp.minimum(m, nmc), 0)),
            pl.BlockSpec((NH * DH, H), lambda m: (0, 0)),
            pl.BlockSpec((1, H), lambda m: (0, 0)),
            pl.BlockSpec((H, I), lambda m: (0, 0)),
            pl.BlockSpec((H, I), lambda m: (0, 0)),
            pl.BlockSpec((I, H), lambda m: (0, 0)),
            pl.BlockSpec((H, E), lambda m: (0, 0)),
        ],
        out_specs=[
            pl.BlockSpec((MB, H), lambda m: (jnp.minimum(m, nmc), 0)),
            pl.BlockSpec((MB, H), lambda m: (jnp.minimum(m, nmc), 0)),
            pl.BlockSpec((S, 1), lambda m: (0, 0)),
            pl.BlockSpec((S, 1), lambda m: (0, 0)),
            pl.BlockSpec((S, 1), lambda m: (0, 0)),
            pl.BlockSpec((S, 1), lambda m: (0, 0)),
            pl.BlockSpec((64, 1), lambda m: (0, 0)),
            pl.BlockSpec((1, 1), lambda m: (0, 0)),
        ],
        out_shape=[
            jax.ShapeDtypeStruct((S, H), f32),
            jax.ShapeDtypeStruct((S, H), f32),
            jax.ShapeDtypeStruct((S, 1), i32),
            jax.ShapeDtypeStruct((S, 1), i32),
            jax.ShapeDtypeStruct((S, 1), f32),
            jax.ShapeDtypeStruct((S, 1), f32),
            jax.ShapeDtypeStruct((64, 1), i32),
            jax.ShapeDtypeStruct((1, 1), i32),
        ],
        scratch_shapes=[
            pltpu.VMEM((S, E), f32),
            pltpu.VMEM((S, E), f32),
            pltpu.VMEM((S, E), f32),
            pltpu.VMEM((1, E), f32),
        ],
    )(o_flat, hs, wo_b, pnw, wgs_b, wus_b, wds_b, W_gate)

    d_all = jnp.concatenate([d_lo, d_hi], axis=0).reshape(NP)
    te = te64.reshape(64)
    nt = ntile.reshape(1)

    x_sorted = _sc_scatter_rows(x2, d_all)

    y_sorted = pl.pallas_call(
        _grouped_ffn_kernel,
        grid_spec=pltpu.PrefetchScalarGridSpec(
            num_scalar_prefetch=2,
            grid=(TMAX,),
            in_specs=[
                pl.BlockSpec((GT, H), lambda j, te, nt: (j, 0)),
                pl.BlockSpec((1, H, I), lambda j, te, nt: (te[j], 0, 0)),
                pl.BlockSpec((1, H, I), lambda j, te, nt: (te[j], 0, 0)),
                pl.BlockSpec((1, I, H), lambda j, te, nt: (te[j], 0, 0)),
            ],
            out_specs=pl.BlockSpec((GT, H), lambda j, te, nt: (j, 0)),
        ),
        out_shape=jax.ShapeDtypeStruct((PAD, H), f32),
    )(te, nt, x_sorted, wge_b, wue_b, wde_b)

    y_pairs = _sc_gather_rows(y_sorted, d_all)
    y1 = y_pairs[:S]
    y2 = y_pairs[S:]

    out = pl.pallas_call(
        _combine_kernel,
        grid=(nm,),
        in_specs=[
            pl.BlockSpec((MB, H), lambda m: (m, 0)),
            pl.BlockSpec((MB, H), lambda m: (m, 0)),
            pl.BlockSpec((MB, H), lambda m: (m, 0)),
            pl.BlockSpec((MB, 1), lambda m: (m, 0)),
            pl.BlockSpec((MB, 1), lambda m: (m, 0)),
        ],
        out_specs=pl.BlockSpec((MB, H), lambda m: (m, 0)),
        out_shape=jax.ShapeDtypeStruct((S, H), f32),
    )(y_base, y1, y2, g_lo, g_hi)

    return out.reshape(B, S, H)
